# Initial kernel scaffold; baseline (speedup 1.0000x reference)
import jax
import jax.numpy as jnp
from jax.experimental import pallas as pl


def kernel(user_emb, item_emb, W1, b1, W2, b2, edge_index):
    raise NotImplementedError("write your pallas kernel here")


# speedup vs baseline: 20.4327x; 20.4327x over previous
import jax
import jax.numpy as jnp
from jax import lax
from jax.experimental import pallas as pl
from jax.experimental.pallas import tpu as pltpu
from jax.experimental.pallas import tpu_sc as plsc

NU = 60000
NI = 40000
NN = NU + NI
EE = 1600000
DD = 32
HD = 16
LL = 3
NC = 2
NS = 16
LANES = 16
NP = 100352
SL = NP // NS
CO = SL // 8
SUB = 128
KS = 8
CE = KS * SUB
EP = NS * CE * 98
EG = EP // SUB
TB = 2048
GRID = NP // TB

f32 = jnp.float32
i32 = jnp.int32


def _sc_mesh():
    return plsc.VectorSubcoreMesh(
        core_axis_name="c", subcore_axis_name="s",
        num_cores=NC, num_subcores=NS)


_SC_PARAMS = pltpu.CompilerParams(use_tc_tiling_on_sc=False)


def _zero_rows(ref, nrows):
    def body(i, carry):
        ref[i, :] = jnp.zeros((LANES,), f32)
        return carry
    lax.fori_loop(0, nrows, body, 0)


def _deg_body(row_hbm, col_hbm, deg_hbm, accum, ridx, cidx, ones, stage):
    c = lax.axis_index("c")
    s = lax.axis_index("s")

    def zf(i, carry):
        stage[pl.ds(pl.multiple_of(i * LANES, LANES), LANES)] = (
            jnp.zeros((LANES,), f32))
        return carry
    lax.fori_loop(0, SL // LANES, zf, 0)
    pltpu.sync_copy(stage, accum.at[pl.ds(pl.multiple_of(s * SL, 8), SL)])

    def of(i, carry):
        ones[pl.ds(pl.multiple_of(i * LANES, LANES), LANES)] = (
            jnp.ones((LANES,), f32))
        return carry
    lax.fori_loop(0, SUB // LANES, of, 0)
    plsc.subcore_barrier()

    epc = EP // (NC * NS)
    gbase = (c * NS + s) * (epc // SUB)

    def chunk(j, carry):
        goff = pl.multiple_of(gbase + j * KS, KS)
        pltpu.sync_copy(row_hbm.at[pl.ds(goff, KS)], ridx)
        pltpu.sync_copy(col_hbm.at[pl.ds(goff, KS)], cidx)
        for k in range(KS):
            pltpu.sync_copy(ones, accum.at[ridx.at[k]], add=True)
            pltpu.sync_copy(ones, accum.at[cidx.at[k]], add=True)
        return carry
    lax.fori_loop(0, epc // CE, chunk, 0)
    plsc.subcore_barrier()

    off = pl.multiple_of(s * SL, 8)
    pltpu.sync_copy(accum.at[pl.ds(off, SL)], stage)
    pltpu.sync_copy(stage, deg_hbm.at[c, pl.ds(off, SL)])


def _spmm_body(emb2_hbm, row_hbm, col_hbm, x_hbm,
               accum, cidx, gidx, ridx, rows, sem):
    c = lax.axis_index("c")
    s = lax.axis_index("s")

    _zero_rows(rows, CO)
    for k in range(SL // CO):
        off = pl.multiple_of(s * SL + k * CO, 8)
        pltpu.sync_copy(rows.at[pl.ds(0, CO)], accum.at[pl.ds(off, CO)])
    plsc.subcore_barrier()

    eps = EP // NS
    gbase = s * (eps // SUB)

    def chunk(j, carry):
        goff = pl.multiple_of(gbase + j * KS, KS)
        pltpu.sync_copy(col_hbm.at[pl.ds(goff, KS)], cidx)

        def tf(i, carry2):
            o = pl.ds(pl.multiple_of(i * LANES, LANES), LANES)
            for k in range(KS):
                gidx[k, o] = cidx[k, o] * 2 + c
            return carry2
        lax.fori_loop(0, SUB // LANES, tf, 0)

        descs = []
        for k in range(KS):
            descs.append(pltpu.async_copy(
                emb2_hbm.at[gidx.at[k]],
                rows.at[pl.ds(k * SUB, SUB)], sem))
        pltpu.sync_copy(row_hbm.at[pl.ds(goff, KS)], ridx)
        for d in descs:
            d.wait()
        for k in range(KS):
            pltpu.sync_copy(rows.at[pl.ds(k * SUB, SUB)],
                            accum.at[ridx.at[k]], add=True)
        return carry
    lax.fori_loop(0, eps // CE, chunk, 0)
    plsc.subcore_barrier()

    for k in range(SL // CO):
        off = pl.multiple_of(s * SL + k * CO, 8)
        pltpu.sync_copy(accum.at[pl.ds(off, CO)], rows.at[pl.ds(0, CO)])
        pltpu.sync_copy(rows.at[pl.ds(0, CO)], x_hbm.at[c, pl.ds(off, CO)])


def _deg_call(row2, col2):
    fn = pl.kernel(
        _deg_body,
        out_type=jax.ShapeDtypeStruct((NC, NP), f32),
        mesh=_sc_mesh(),
        compiler_params=_SC_PARAMS,
        scratch_types=[
            pltpu.VMEM_SHARED((NP,), f32),
            pltpu.VMEM((KS, SUB), i32),
            pltpu.VMEM((KS, SUB), i32),
            pltpu.VMEM((SUB,), f32),
            pltpu.VMEM((SL,), f32),
        ],
    )
    return fn(row2, col2)


def _spmm_call(emb2, row2, col2):
    fn = pl.kernel(
        _spmm_body,
        out_type=jax.ShapeDtypeStruct((NC, NP, HD), f32),
        mesh=_sc_mesh(),
        compiler_params=_SC_PARAMS,
        scratch_types=[
            pltpu.VMEM_SHARED((NP, HD), f32),
            pltpu.VMEM((KS, SUB), i32),
            pltpu.VMEM((KS, SUB), i32),
            pltpu.VMEM((KS, SUB), i32),
            pltpu.VMEM((CE, HD), f32),
            pltpu.SemaphoreType.DMA,
        ],
    )
    return fn(emb2, row2, col2)


def _prep_tc(degp3, emb0):
    def body(dp_ref, e_ref, dis_ref, es_ref):
        dp = dp_ref[...]
        deg = dp[0] + dp[1]
        dis = lax.rsqrt(jnp.maximum(deg, 1.0))
        dis_ref[...] = dis
        es_ref[...] = e_ref[...] * dis

    return pl.pallas_call(
        body,
        grid=(GRID,),
        in_specs=[
            pl.BlockSpec((NC, TB, 1), lambda i: (0, i, 0)),
            pl.BlockSpec((TB, DD), lambda i: (i, 0)),
        ],
        out_specs=[
            pl.BlockSpec((TB, 1), lambda i: (i, 0)),
            pl.BlockSpec((TB, DD), lambda i: (i, 0)),
        ],
        out_shape=[
            jax.ShapeDtypeStruct((NP, 1), f32),
            jax.ShapeDtypeStruct((NP, DD), f32),
        ],
    )(degp3, emb0)


def _layer_tc(emb, x2, dis, w1t, b1l, w2t, b2l):
    def body(e_ref, xa_ref, xb_ref, dis_ref, w1_ref, b1_ref, w2_ref, b2_ref,
             eo_ref, eso_ref):
        dis_blk = dis_ref[...]
        x = jnp.concatenate([xa_ref[0], xb_ref[0]], axis=1) * dis_blk
        e = e_ref[...]
        h = (jnp.dot(e + x, w1_ref[...], preferred_element_type=f32)
             + b1_ref[...]
             + jnp.dot(x * e, w2_ref[...], preferred_element_type=f32)
             + b2_ref[...])
        a = jnp.where(h >= 0, h, 0.2 * h)
        nrm = jnp.maximum(jnp.sqrt(jnp.sum(a * a, axis=1, keepdims=True)),
                          1e-12)
        o = a / nrm
        eo_ref[...] = o
        eso_ref[...] = o * dis_blk

    return pl.pallas_call(
        body,
        grid=(GRID,),
        in_specs=[
            pl.BlockSpec((TB, DD), lambda i: (i, 0)),
            pl.BlockSpec((1, TB, HD), lambda i: (0, i, 0)),
            pl.BlockSpec((1, TB, HD), lambda i: (1, i, 0)),
            pl.BlockSpec((TB, 1), lambda i: (i, 0)),
            pl.BlockSpec((DD, DD), lambda i: (0, 0)),
            pl.BlockSpec((1, DD), lambda i: (0, 0)),
            pl.BlockSpec((DD, DD), lambda i: (0, 0)),
            pl.BlockSpec((1, DD), lambda i: (0, 0)),
        ],
        out_specs=[
            pl.BlockSpec((TB, DD), lambda i: (i, 0)),
            pl.BlockSpec((TB, DD), lambda i: (i, 0)),
        ],
        out_shape=[
            jax.ShapeDtypeStruct((NP, DD), f32),
            jax.ShapeDtypeStruct((NP, DD), f32),
        ],
    )(emb, x2, x2, dis, w1t, b1l, w2t, b2l)


def kernel(user_emb, item_emb, W1, b1, W2, b2, edge_index):
    edge_index = edge_index.astype(i32)
    row2 = jnp.pad(edge_index[0], (0, EP - EE),
                   constant_values=NN).reshape(EG, SUB)
    col2 = jnp.pad(edge_index[1], (0, EP - EE),
                   constant_values=NN).reshape(EG, SUB)

    emb0 = jnp.concatenate([user_emb, item_emb], axis=0)
    emb0 = jnp.pad(emb0, ((0, NP - NN), (0, 0)))

    degp = _deg_call(row2, col2)
    dis, embs = _prep_tc(degp.reshape(NC, NP, 1), emb0)

    outs = [emb0]
    emb = emb0
    for l in range(LL):
        x2 = _spmm_call(embs.reshape(2 * NP, HD), row2, col2)
        emb, embs = _layer_tc(emb, x2, dis,
                              W1[l].T, b1[l][None], W2[l].T, b2[l][None])
        outs.append(emb)

    alle = jnp.concatenate(outs, axis=1)[:NN]
    return (alle[:NU], alle[NU:])


# speedup vs baseline: 20.5234x; 1.0044x over previous
import jax
import jax.numpy as jnp
from jax import lax
from jax.experimental import pallas as pl
from jax.experimental.pallas import tpu as pltpu
from jax.experimental.pallas import tpu_sc as plsc

NU = 60000
NI = 40000
NN = NU + NI
EE = 1600000
DD = 32
HD = 16
LL = 3
NC = 2
NS = 16
LANES = 16
NP = 100352
SL = NP // NS
CO = SL // 16
SUB = 128
KS = 4
CE = KS * SUB
EP = NS * CE * 196
EG = EP // SUB
TB = 2048
GRID = NP // TB

f32 = jnp.float32
i32 = jnp.int32


def _sc_mesh():
    return plsc.VectorSubcoreMesh(
        core_axis_name="c", subcore_axis_name="s",
        num_cores=NC, num_subcores=NS)


_SC_PARAMS = pltpu.CompilerParams(use_tc_tiling_on_sc=False)


def _zero_rows(ref, nrows):
    def body(i, carry):
        ref[i, :] = jnp.zeros((LANES,), f32)
        return carry
    lax.fori_loop(0, nrows, body, 0)


def _deg_body(row_hbm, col_hbm, deg_hbm, accum, ridx, cidx, ones, stage):
    c = lax.axis_index("c")
    s = lax.axis_index("s")

    def zf(i, carry):
        stage[pl.ds(pl.multiple_of(i * LANES, LANES), LANES)] = (
            jnp.zeros((LANES,), f32))
        return carry
    lax.fori_loop(0, SL // LANES, zf, 0)
    pltpu.sync_copy(stage, accum.at[pl.ds(pl.multiple_of(s * SL, 8), SL)])

    def of(i, carry):
        ones[pl.ds(pl.multiple_of(i * LANES, LANES), LANES)] = (
            jnp.ones((LANES,), f32))
        return carry
    lax.fori_loop(0, SUB // LANES, of, 0)
    plsc.subcore_barrier()

    epc = EP // (NC * NS)
    gbase = (c * NS + s) * (epc // SUB)

    def chunk(j, carry):
        goff = pl.multiple_of(gbase + j * KS, KS)
        pltpu.sync_copy(row_hbm.at[pl.ds(goff, KS)], ridx)
        pltpu.sync_copy(col_hbm.at[pl.ds(goff, KS)], cidx)
        for k in range(KS):
            pltpu.sync_copy(ones, accum.at[ridx.at[k]], add=True)
            pltpu.sync_copy(ones, accum.at[cidx.at[k]], add=True)
        return carry
    lax.fori_loop(0, epc // CE, chunk, 0)
    plsc.subcore_barrier()

    off = pl.multiple_of(s * SL, 8)
    pltpu.sync_copy(accum.at[pl.ds(off, SL)], stage)
    pltpu.sync_copy(stage, deg_hbm.at[c, pl.ds(off, SL)])


def _spmm_body(emb2_hbm, row_hbm, col_hbm, x_hbm,
               accum, cidx0, gidx0, ridx0, rows0, cidx1, gidx1, ridx1, rows1,
               gsem0, gsem1, ssem0, ssem1):
    c = lax.axis_index("c")
    s = lax.axis_index("s")

    _zero_rows(rows0, CO)
    for k in range(SL // CO):
        off = pl.multiple_of(s * SL + k * CO, 8)
        pltpu.sync_copy(rows0.at[pl.ds(0, CO)], accum.at[pl.ds(off, CO)])
    plsc.subcore_barrier()

    eps = EP // NS
    gbase = s * (eps // SUB)
    bufs = ((cidx0, gidx0, ridx0, rows0, gsem0, ssem0),
            (cidx1, gidx1, ridx1, rows1, gsem1, ssem1))

    def stage(j, buf):
        cidx, gidx, ridx, rows, gsem, _ = buf
        goff = pl.multiple_of(gbase + j * KS, KS)
        pltpu.sync_copy(col_hbm.at[pl.ds(goff, KS)], cidx)

        def tf(i, carry2):
            o = pl.ds(pl.multiple_of(i * LANES, LANES), LANES)
            for k in range(KS):
                gidx[k, o] = cidx[k, o] * 2 + c
            return carry2
        lax.fori_loop(0, SUB // LANES, tf, 0)
        pltpu.sync_copy(row_hbm.at[pl.ds(goff, KS)], ridx)
        for k in range(KS):
            pltpu.async_copy(emb2_hbm.at[gidx.at[k]],
                             rows.at[pl.ds(k * SUB, SUB)], gsem)

    def drain_gathers(buf):
        _, _, _, rows, gsem, _ = buf
        pltpu.make_async_copy(emb2_hbm.at[pl.ds(0, CE)], rows, gsem).wait()

    def drain_scatters(buf):
        _, _, _, rows, _, ssem = buf
        pltpu.make_async_copy(emb2_hbm.at[pl.ds(0, CE)], rows, ssem).wait()

    def scatter(buf):
        _, _, ridx, rows, _, ssem = buf
        for k in range(KS):
            pltpu.async_copy(rows.at[pl.ds(k * SUB, SUB)],
                             accum.at[ridx.at[k]], ssem, add=True)

    def pair(g, carry):
        pl.when(g > 0)(lambda: drain_scatters(bufs[0]))
        stage(2 * g, bufs[0])
        pl.when(g > 0)(lambda: drain_scatters(bufs[1]))
        stage(2 * g + 1, bufs[1])
        drain_gathers(bufs[0])
        scatter(bufs[0])
        drain_gathers(bufs[1])
        scatter(bufs[1])
        return carry
    lax.fori_loop(0, eps // CE // 2, pair, 0)
    drain_scatters(bufs[0])
    drain_scatters(bufs[1])
    plsc.subcore_barrier()

    for k in range(SL // CO):
        off = pl.multiple_of(s * SL + k * CO, 8)
        pltpu.sync_copy(accum.at[pl.ds(off, CO)], rows0.at[pl.ds(0, CO)])
        pltpu.sync_copy(rows0.at[pl.ds(0, CO)], x_hbm.at[c, pl.ds(off, CO)])


def _deg_call(row2, col2):
    fn = pl.kernel(
        _deg_body,
        out_type=jax.ShapeDtypeStruct((NC, NP), f32),
        mesh=_sc_mesh(),
        compiler_params=_SC_PARAMS,
        scratch_types=[
            pltpu.VMEM_SHARED((NP,), f32),
            pltpu.VMEM((KS, SUB), i32),
            pltpu.VMEM((KS, SUB), i32),
            pltpu.VMEM((SUB,), f32),
            pltpu.VMEM((SL,), f32),
        ],
    )
    return fn(row2, col2)


def _spmm_call(emb2, row2, col2):
    fn = pl.kernel(
        _spmm_body,
        out_type=jax.ShapeDtypeStruct((NC, NP, HD), f32),
        mesh=_sc_mesh(),
        compiler_params=_SC_PARAMS,
        scratch_types=[
            pltpu.VMEM_SHARED((NP, HD), f32),
            pltpu.VMEM((KS, SUB), i32),
            pltpu.VMEM((KS, SUB), i32),
            pltpu.VMEM((KS, SUB), i32),
            pltpu.VMEM((CE, HD), f32),
            pltpu.VMEM((KS, SUB), i32),
            pltpu.VMEM((KS, SUB), i32),
            pltpu.VMEM((KS, SUB), i32),
            pltpu.VMEM((CE, HD), f32),
            pltpu.SemaphoreType.DMA,
            pltpu.SemaphoreType.DMA,
            pltpu.SemaphoreType.DMA,
            pltpu.SemaphoreType.DMA,
        ],
    )
    return fn(emb2, row2, col2)


def _prep_tc(degp3, emb0):
    def body(dp_ref, e_ref, dis_ref, es_ref):
        dp = dp_ref[...]
        deg = dp[0] + dp[1]
        dis = lax.rsqrt(jnp.maximum(deg, 1.0))
        dis_ref[...] = dis
        es_ref[...] = e_ref[...] * dis

    return pl.pallas_call(
        body,
        grid=(GRID,),
        in_specs=[
            pl.BlockSpec((NC, TB, 1), lambda i: (0, i, 0)),
            pl.BlockSpec((TB, DD), lambda i: (i, 0)),
        ],
        out_specs=[
            pl.BlockSpec((TB, 1), lambda i: (i, 0)),
            pl.BlockSpec((TB, DD), lambda i: (i, 0)),
        ],
        out_shape=[
            jax.ShapeDtypeStruct((NP, 1), f32),
            jax.ShapeDtypeStruct((NP, DD), f32),
        ],
    )(degp3, emb0)


def _layer_tc(emb, x2, dis, w1t, b1l, w2t, b2l):
    def body(e_ref, xa_ref, xb_ref, dis_ref, w1_ref, b1_ref, w2_ref, b2_ref,
             eo_ref, eso_ref):
        dis_blk = dis_ref[...]
        x = jnp.concatenate([xa_ref[0], xb_ref[0]], axis=1) * dis_blk
        e = e_ref[...]
        h = (jnp.dot(e + x, w1_ref[...], preferred_element_type=f32)
             + b1_ref[...]
             + jnp.dot(x * e, w2_ref[...], preferred_element_type=f32)
             + b2_ref[...])
        a = jnp.where(h >= 0, h, 0.2 * h)
        nrm = jnp.maximum(jnp.sqrt(jnp.sum(a * a, axis=1, keepdims=True)),
                          1e-12)
        o = a / nrm
        eo_ref[...] = o
        eso_ref[...] = o * dis_blk

    return pl.pallas_call(
        body,
        grid=(GRID,),
        in_specs=[
            pl.BlockSpec((TB, DD), lambda i: (i, 0)),
            pl.BlockSpec((1, TB, HD), lambda i: (0, i, 0)),
            pl.BlockSpec((1, TB, HD), lambda i: (1, i, 0)),
            pl.BlockSpec((TB, 1), lambda i: (i, 0)),
            pl.BlockSpec((DD, DD), lambda i: (0, 0)),
            pl.BlockSpec((1, DD), lambda i: (0, 0)),
            pl.BlockSpec((DD, DD), lambda i: (0, 0)),
            pl.BlockSpec((1, DD), lambda i: (0, 0)),
        ],
        out_specs=[
            pl.BlockSpec((TB, DD), lambda i: (i, 0)),
            pl.BlockSpec((TB, DD), lambda i: (i, 0)),
        ],
        out_shape=[
            jax.ShapeDtypeStruct((NP, DD), f32),
            jax.ShapeDtypeStruct((NP, DD), f32),
        ],
    )(emb, x2, x2, dis, w1t, b1l, w2t, b2l)


def kernel(user_emb, item_emb, W1, b1, W2, b2, edge_index):
    edge_index = edge_index.astype(i32)
    row2 = jnp.pad(edge_index[0], (0, EP - EE),
                   constant_values=NN).reshape(EG, SUB)
    col2 = jnp.pad(edge_index[1], (0, EP - EE),
                   constant_values=NN).reshape(EG, SUB)

    emb0 = jnp.concatenate([user_emb, item_emb], axis=0)
    emb0 = jnp.pad(emb0, ((0, NP - NN), (0, 0)))

    degp = _deg_call(row2, col2)
    dis, embs = _prep_tc(degp.reshape(NC, NP, 1), emb0)

    outs = [emb0]
    emb = emb0
    for l in range(LL):
        x2 = _spmm_call(embs.reshape(2 * NP, HD), row2, col2)
        emb, embs = _layer_tc(emb, x2, dis,
                              W1[l].T, b1[l][None], W2[l].T, b2[l][None])
        outs.append(emb)

    alle = jnp.concatenate(outs, axis=1)[:NN]
    return (alle[:NU], alle[NU:])


# speedup vs baseline: 25.8838x; 1.2612x over previous
import jax
import jax.numpy as jnp
from jax import lax
from jax.experimental import pallas as pl
from jax.experimental.pallas import tpu as pltpu
from jax.experimental.pallas import tpu_sc as plsc

NU = 60000
NI = 40000
NN = NU + NI
EE = 1600000
DD = 32
LL = 3
NC = 2
NS = 16
LANES = 16
NP = 100352
SL = NP // NS
CO = SL // 16
SUB = 128
KS = 4
CE = KS * SUB
EP = NS * CE * 196
EG = EP // SUB
TB = 2048
GRID = NP // TB

f32 = jnp.float32
bf16 = jnp.bfloat16
i32 = jnp.int32


def _sc_mesh():
    return plsc.VectorSubcoreMesh(
        core_axis_name="c", subcore_axis_name="s",
        num_cores=NC, num_subcores=NS)


_SC_PARAMS = pltpu.CompilerParams(use_tc_tiling_on_sc=False)


def _deg_body(row_hbm, col_hbm, deg_hbm, accum, ridx, cidx, ones, stage):
    c = lax.axis_index("c")
    s = lax.axis_index("s")

    def zf(i, carry):
        stage[pl.ds(pl.multiple_of(i * LANES, LANES), LANES)] = (
            jnp.zeros((LANES,), f32))
        return carry
    lax.fori_loop(0, SL // LANES, zf, 0)
    pltpu.sync_copy(stage, accum.at[pl.ds(pl.multiple_of(s * SL, 8), SL)])

    def of(i, carry):
        ones[pl.ds(pl.multiple_of(i * LANES, LANES), LANES)] = (
            jnp.ones((LANES,), f32))
        return carry
    lax.fori_loop(0, SUB // LANES, of, 0)
    plsc.subcore_barrier()

    epc = EP // (NC * NS)
    dkv = 2
    gbase = (c * NS + s) * (epc // SUB)

    def chunk(j, carry):
        goff = pl.multiple_of(gbase + j * KS * dkv, KS)
        pltpu.sync_copy(row_hbm.at[pl.ds(goff, KS * dkv)], ridx)
        pltpu.sync_copy(col_hbm.at[pl.ds(goff, KS * dkv)], cidx)
        for k in range(KS * dkv):
            pltpu.sync_copy(ones, accum.at[ridx.at[k]], add=True)
            pltpu.sync_copy(ones, accum.at[cidx.at[k]], add=True)
        return carry
    lax.fori_loop(0, epc // (CE * dkv), chunk, 0)
    plsc.subcore_barrier()

    off = pl.multiple_of(s * SL, 8)
    pltpu.sync_copy(accum.at[pl.ds(off, SL)], stage)
    pltpu.sync_copy(stage, deg_hbm.at[c, pl.ds(off, SL)])


def _zero_rows_bf(ref, nrows):
    def body(i, carry):
        ref[i, :] = jnp.zeros((DD,), bf16)
        return carry
    lax.fori_loop(0, nrows, body, 0)


def _spmm_body(embbf_hbm, row_hbm, col_hbm, x_hbm,
               accum, cidx0, ridx0, rows0, cidx1, ridx1, rows1,
               gsem0, gsem1, ssem0, ssem1):
    c = lax.axis_index("c")
    s = lax.axis_index("s")

    _zero_rows_bf(rows0, CO)
    for k in range(SL // CO):
        off = pl.multiple_of(s * SL + k * CO, 8)
        pltpu.sync_copy(rows0.at[pl.ds(0, CO)], accum.at[pl.ds(off, CO)])
    plsc.subcore_barrier()

    epc = EP // (NC * NS)
    gbase = (c * NS + s) * (epc // SUB)
    bufs = ((cidx0, ridx0, rows0, gsem0, ssem0),
            (cidx1, ridx1, rows1, gsem1, ssem1))

    def stage(j, buf):
        cidx, ridx, rows, gsem, _ = buf
        goff = pl.multiple_of(gbase + j * KS, KS)
        pltpu.sync_copy(col_hbm.at[pl.ds(goff, KS)], cidx)
        pltpu.sync_copy(row_hbm.at[pl.ds(goff, KS)], ridx)
        for k in range(KS):
            pltpu.async_copy(embbf_hbm.at[cidx.at[k]],
                             rows.at[pl.ds(k * SUB, SUB)], gsem)

    def drain(buf, which):
        _, _, rows, gsem, ssem = buf
        sem = gsem if which == 0 else ssem
        pltpu.make_async_copy(embbf_hbm.at[pl.ds(0, CE)], rows, sem).wait()

    def scatter(buf):
        _, ridx, rows, _, ssem = buf
        for k in range(KS):
            pltpu.async_copy(rows.at[pl.ds(k * SUB, SUB)],
                             accum.at[ridx.at[k]], ssem, add=True)

    def pair(g, carry):
        pl.when(g > 0)(lambda: drain(bufs[0], 1))
        stage(2 * g, bufs[0])
        pl.when(g > 0)(lambda: drain(bufs[1], 1))
        stage(2 * g + 1, bufs[1])
        drain(bufs[0], 0)
        scatter(bufs[0])
        drain(bufs[1], 0)
        scatter(bufs[1])
        return carry
    lax.fori_loop(0, epc // CE // 2, pair, 0)
    drain(bufs[0], 1)
    drain(bufs[1], 1)
    plsc.subcore_barrier()

    for k in range(SL // CO):
        off = pl.multiple_of(s * SL + k * CO, 8)
        pltpu.sync_copy(accum.at[pl.ds(off, CO)], rows0.at[pl.ds(0, CO)])
        pltpu.sync_copy(rows0.at[pl.ds(0, CO)], x_hbm.at[c, pl.ds(off, CO)])


def _deg_call(row2, col2):
    fn = pl.kernel(
        _deg_body,
        out_type=jax.ShapeDtypeStruct((NC, NP), f32),
        mesh=_sc_mesh(),
        compiler_params=_SC_PARAMS,
        scratch_types=[
            pltpu.VMEM_SHARED((NP,), f32),
            pltpu.VMEM((2 * KS, SUB), i32),
            pltpu.VMEM((2 * KS, SUB), i32),
            pltpu.VMEM((SUB,), f32),
            pltpu.VMEM((SL,), f32),
        ],
    )
    return fn(row2, col2)


def _spmm_call(embbf, row2, col2):
    fn = pl.kernel(
        _spmm_body,
        out_type=jax.ShapeDtypeStruct((NC, NP, DD), bf16),
        mesh=_sc_mesh(),
        compiler_params=_SC_PARAMS,
        scratch_types=[
            pltpu.VMEM_SHARED((NP, DD), bf16),
            pltpu.VMEM((KS, SUB), i32),
            pltpu.VMEM((KS, SUB), i32),
            pltpu.VMEM((CE, DD), bf16),
            pltpu.VMEM((KS, SUB), i32),
            pltpu.VMEM((KS, SUB), i32),
            pltpu.VMEM((CE, DD), bf16),
            pltpu.SemaphoreType.DMA,
            pltpu.SemaphoreType.DMA,
            pltpu.SemaphoreType.DMA,
            pltpu.SemaphoreType.DMA,
        ],
    )
    return fn(embbf, row2, col2)


def _prep_tc(degp3, emb0):
    def body(dp_ref, e_ref, dis_ref, es_ref):
        dp = dp_ref[...]
        deg = dp[0] + dp[1]
        dis = lax.rsqrt(jnp.maximum(deg, 1.0))
        dis_ref[...] = dis
        es_ref[...] = (e_ref[...] * dis).astype(bf16)

    return pl.pallas_call(
        body,
        grid=(GRID,),
        in_specs=[
            pl.BlockSpec((NC, TB, 1), lambda i: (0, i, 0)),
            pl.BlockSpec((TB, DD), lambda i: (i, 0)),
        ],
        out_specs=[
            pl.BlockSpec((TB, 1), lambda i: (i, 0)),
            pl.BlockSpec((TB, DD), lambda i: (i, 0)),
        ],
        out_shape=[
            jax.ShapeDtypeStruct((NP, 1), f32),
            jax.ShapeDtypeStruct((NP, DD), bf16),
        ],
    )(degp3, emb0)


def _layer_tc(emb, x2, dis, w1t, b1l, w2t, b2l):
    def body(e_ref, xa_ref, xb_ref, dis_ref, w1_ref, b1_ref, w2_ref, b2_ref,
             eo_ref, eso_ref):
        dis_blk = dis_ref[...]
        x = (xa_ref[0].astype(f32) + xb_ref[0].astype(f32)) * dis_blk
        e = e_ref[...]
        h = (jnp.dot(e + x, w1_ref[...], preferred_element_type=f32)
             + b1_ref[...]
             + jnp.dot(x * e, w2_ref[...], preferred_element_type=f32)
             + b2_ref[...])
        a = jnp.where(h >= 0, h, 0.2 * h)
        nrm = jnp.maximum(jnp.sqrt(jnp.sum(a * a, axis=1, keepdims=True)),
                          1e-12)
        o = a / nrm
        eo_ref[...] = o
        eso_ref[...] = (o * dis_blk).astype(bf16)

    return pl.pallas_call(
        body,
        grid=(GRID,),
        in_specs=[
            pl.BlockSpec((TB, DD), lambda i: (i, 0)),
            pl.BlockSpec((1, TB, DD), lambda i: (0, i, 0)),
            pl.BlockSpec((1, TB, DD), lambda i: (1, i, 0)),
            pl.BlockSpec((TB, 1), lambda i: (i, 0)),
            pl.BlockSpec((DD, DD), lambda i: (0, 0)),
            pl.BlockSpec((1, DD), lambda i: (0, 0)),
            pl.BlockSpec((DD, DD), lambda i: (0, 0)),
            pl.BlockSpec((1, DD), lambda i: (0, 0)),
        ],
        out_specs=[
            pl.BlockSpec((TB, DD), lambda i: (i, 0)),
            pl.BlockSpec((TB, DD), lambda i: (i, 0)),
        ],
        out_shape=[
            jax.ShapeDtypeStruct((NP, DD), f32),
            jax.ShapeDtypeStruct((NP, DD), bf16),
        ],
    )(emb, x2, x2, dis, w1t, b1l, w2t, b2l)


def kernel(user_emb, item_emb, W1, b1, W2, b2, edge_index):
    edge_index = edge_index.astype(i32)
    row2 = jnp.pad(edge_index[0], (0, EP - EE),
                   constant_values=NN).reshape(EG, SUB)
    col2 = jnp.pad(edge_index[1], (0, EP - EE),
                   constant_values=NN).reshape(EG, SUB)

    emb0 = jnp.concatenate([user_emb, item_emb], axis=0)
    emb0 = jnp.pad(emb0, ((0, NP - NN), (0, 0)))

    degp = _deg_call(row2, col2)
    dis, embbf = _prep_tc(degp.reshape(NC, NP, 1), emb0)

    outs = [emb0]
    emb = emb0
    for l in range(LL):
        x2 = _spmm_call(embbf, row2, col2)
        emb, embbf = _layer_tc(emb, x2, dis,
                               W1[l].T, b1[l][None], W2[l].T, b2[l][None])
        outs.append(emb)

    alle = jnp.concatenate(outs, axis=1)[:NN]
    return (alle[:NU], alle[NU:])


# speedup vs baseline: 27.9105x; 1.0783x over previous
import jax
import jax.numpy as jnp
from jax import lax
from jax.experimental import pallas as pl
from jax.experimental.pallas import tpu as pltpu
from jax.experimental.pallas import tpu_sc as plsc

NU = 60000
NI = 40000
NN = NU + NI
EE = 1600000
DD = 32
LL = 3
NC = 2
NS = 16
LANES = 16
NP = 100352
SL = NP // NS
CO = SL // 16
SUB = 128
KS = 4
CE = KS * SUB
EP = NS * CE * 196
EG = EP // SUB
TB = 2048
GRID = NP // TB

f32 = jnp.float32
bf16 = jnp.bfloat16
i32 = jnp.int32


def _sc_mesh():
    return plsc.VectorSubcoreMesh(
        core_axis_name="c", subcore_axis_name="s",
        num_cores=NC, num_subcores=NS)


_SC_PARAMS = pltpu.CompilerParams(use_tc_tiling_on_sc=False)


def _deg_body(row_hbm, col_hbm, deg_hbm, accum, ridx, cidx, ones, stage):
    c = lax.axis_index("c")
    s = lax.axis_index("s")

    def zf(i, carry):
        stage[pl.ds(pl.multiple_of(i * LANES, LANES), LANES)] = (
            jnp.zeros((LANES,), f32))
        return carry
    lax.fori_loop(0, SL // LANES, zf, 0)
    pltpu.sync_copy(stage, accum.at[pl.ds(pl.multiple_of(s * SL, 8), SL)])

    def of(i, carry):
        ones[pl.ds(pl.multiple_of(i * LANES, LANES), LANES)] = (
            jnp.ones((LANES,), f32))
        return carry
    lax.fori_loop(0, SUB // LANES, of, 0)
    plsc.subcore_barrier()

    epc = EP // (NC * NS)
    dkv = 2
    gbase = (c * NS + s) * (epc // SUB)

    def chunk(j, carry):
        goff = pl.multiple_of(gbase + j * KS * dkv, KS)
        pltpu.sync_copy(row_hbm.at[pl.ds(goff, KS * dkv)], ridx)
        pltpu.sync_copy(col_hbm.at[pl.ds(goff, KS * dkv)], cidx)
        for k in range(KS * dkv):
            pltpu.sync_copy(ones, accum.at[ridx.at[k]], add=True)
            pltpu.sync_copy(ones, accum.at[cidx.at[k]], add=True)
        return carry
    lax.fori_loop(0, epc // (CE * dkv), chunk, 0)
    plsc.subcore_barrier()

    off = pl.multiple_of(s * SL, 8)
    pltpu.sync_copy(accum.at[pl.ds(off, SL)], stage)
    pltpu.sync_copy(stage, deg_hbm.at[c, pl.ds(off, SL)])


def _zero_rows_bf(ref, nrows):
    def body(i, carry):
        ref[i, :] = jnp.zeros((DD,), bf16)
        return carry
    lax.fori_loop(0, nrows, body, 0)


def _spmm_body(embbf_hbm, row_hbm, col_hbm, x_hbm,
               accum, cidx0, ridx0, rows0, cidx1, ridx1, rows1,
               gsem0, gsem1, ssem0, ssem1):
    c = lax.axis_index("c")
    s = lax.axis_index("s")

    _zero_rows_bf(rows0, CO)
    for k in range(SL // CO):
        off = pl.multiple_of(s * SL + k * CO, 8)
        pltpu.sync_copy(rows0.at[pl.ds(0, CO)], accum.at[pl.ds(off, CO)])
    plsc.subcore_barrier()

    epc = EP // (NC * NS)
    gbase = (c * NS + s) * (epc // SUB)
    bufs = ((cidx0, ridx0, rows0, gsem0, ssem0),
            (cidx1, ridx1, rows1, gsem1, ssem1))

    def stage(j, buf):
        cidx, ridx, rows, gsem, _ = buf
        goff = pl.multiple_of(gbase + j * KS, KS)
        pltpu.sync_copy(col_hbm.at[pl.ds(goff, KS)], cidx)
        pltpu.sync_copy(row_hbm.at[pl.ds(goff, KS)], ridx)
        for k in range(KS):
            pltpu.async_copy(embbf_hbm.at[cidx.at[k]],
                             rows.at[pl.ds(k * SUB, SUB)], gsem)

    def drain(buf, which):
        _, _, rows, gsem, ssem = buf
        sem = gsem if which == 0 else ssem
        pltpu.make_async_copy(embbf_hbm.at[pl.ds(0, CE)], rows, sem).wait()

    def scatter(buf):
        _, ridx, rows, _, ssem = buf
        for k in range(KS):
            pltpu.async_copy(rows.at[pl.ds(k * SUB, SUB)],
                             accum.at[ridx.at[k]], ssem, add=True)

    def pair(g, carry):
        pl.when(g > 0)(lambda: drain(bufs[0], 1))
        stage(2 * g, bufs[0])
        pl.when(g > 0)(lambda: drain(bufs[1], 1))
        stage(2 * g + 1, bufs[1])
        drain(bufs[0], 0)
        scatter(bufs[0])
        drain(bufs[1], 0)
        scatter(bufs[1])
        return carry
    lax.fori_loop(0, epc // CE // 2, pair, 0)
    drain(bufs[0], 1)
    drain(bufs[1], 1)
    plsc.subcore_barrier()

    for k in range(SL // CO):
        off = pl.multiple_of(s * SL + k * CO, 8)
        pltpu.sync_copy(accum.at[pl.ds(off, CO)], rows0.at[pl.ds(0, CO)])
        pltpu.sync_copy(rows0.at[pl.ds(0, CO)], x_hbm.at[c, pl.ds(off, CO)])


def _deg_call(row2, col2):
    fn = pl.kernel(
        _deg_body,
        out_type=jax.ShapeDtypeStruct((NC, NP), f32),
        mesh=_sc_mesh(),
        compiler_params=_SC_PARAMS,
        scratch_types=[
            pltpu.VMEM_SHARED((NP,), f32),
            pltpu.VMEM((2 * KS, SUB), i32),
            pltpu.VMEM((2 * KS, SUB), i32),
            pltpu.VMEM((SUB,), f32),
            pltpu.VMEM((SL,), f32),
        ],
    )
    return fn(row2, col2)


def _spmm_call(embbf, row2, col2):
    fn = pl.kernel(
        _spmm_body,
        out_type=jax.ShapeDtypeStruct((NC, NP, DD), bf16),
        mesh=_sc_mesh(),
        compiler_params=_SC_PARAMS,
        scratch_types=[
            pltpu.VMEM_SHARED((NP, DD), bf16),
            pltpu.VMEM((KS, SUB), i32),
            pltpu.VMEM((KS, SUB), i32),
            pltpu.VMEM((CE, DD), bf16),
            pltpu.VMEM((KS, SUB), i32),
            pltpu.VMEM((KS, SUB), i32),
            pltpu.VMEM((CE, DD), bf16),
            pltpu.SemaphoreType.DMA,
            pltpu.SemaphoreType.DMA,
            pltpu.SemaphoreType.DMA,
            pltpu.SemaphoreType.DMA,
        ],
    )
    return fn(embbf, row2, col2)


def _dis_tc(degp_packed):
    def body(dp_ref, dis_ref):
        dp = dp_ref[...]
        dis_ref[...] = lax.rsqrt(jnp.maximum(dp[0] + dp[1], 1.0))

    return pl.pallas_call(
        body,
        grid=(GRID,),
        in_specs=[pl.BlockSpec((NC, TB // 128, 128),
                               lambda i: (0, i, 0))],
        out_specs=pl.BlockSpec((TB // 128, 128), lambda i: (i, 0)),
        out_shape=jax.ShapeDtypeStruct((NP // 128, 128), f32),
    )(degp_packed)


def _scale_tc(emb0, dis32):
    def body(e_ref, d_ref, es_ref):
        es_ref[...] = (e_ref[...] * d_ref[...]).astype(bf16)

    return pl.pallas_call(
        body,
        grid=(GRID,),
        in_specs=[
            pl.BlockSpec((TB, DD), lambda i: (i, 0)),
            pl.BlockSpec((TB, DD), lambda i: (i, 0)),
        ],
        out_specs=pl.BlockSpec((TB, DD), lambda i: (i, 0)),
        out_shape=jax.ShapeDtypeStruct((NP, DD), bf16),
    )(emb0, dis32)


def _layer_math(e, xa, xb, dis_blk, w1, b1, w2, b2):
    x = (xa.astype(f32) + xb.astype(f32)) * dis_blk
    h = (jnp.dot(e + x, w1, preferred_element_type=f32) + b1
         + jnp.dot(x * e, w2, preferred_element_type=f32) + b2)
    a = jnp.where(h >= 0, h, 0.2 * h)
    nrm = jnp.maximum(jnp.sqrt(jnp.sum(a * a, axis=1, keepdims=True)), 1e-12)
    return a / nrm


_LAYER_SPECS = [
    pl.BlockSpec((TB, DD), lambda i: (i, 0)),
    pl.BlockSpec((1, TB, DD), lambda i: (0, i, 0)),
    pl.BlockSpec((1, TB, DD), lambda i: (1, i, 0)),
    pl.BlockSpec((TB, DD), lambda i: (i, 0)),
    pl.BlockSpec((DD, DD), lambda i: (0, 0)),
    pl.BlockSpec((1, DD), lambda i: (0, 0)),
    pl.BlockSpec((DD, DD), lambda i: (0, 0)),
    pl.BlockSpec((1, DD), lambda i: (0, 0)),
]


def _layer_tc(emb, x2, dis32, w1t, b1l, w2t, b2l):
    def body(e_ref, xa_ref, xb_ref, dis_ref, w1_ref, b1_ref, w2_ref, b2_ref,
             eo_ref, eso_ref):
        dis_blk = dis_ref[...]
        o = _layer_math(e_ref[...], xa_ref[0], xb_ref[0], dis_blk,
                        w1_ref[...], b1_ref[...], w2_ref[...], b2_ref[...])
        eo_ref[...] = o
        eso_ref[...] = (o * dis_blk).astype(bf16)

    return pl.pallas_call(
        body,
        grid=(GRID,),
        in_specs=_LAYER_SPECS,
        out_specs=[
            pl.BlockSpec((TB, DD), lambda i: (i, 0)),
            pl.BlockSpec((TB, DD), lambda i: (i, 0)),
        ],
        out_shape=[
            jax.ShapeDtypeStruct((NP, DD), f32),
            jax.ShapeDtypeStruct((NP, DD), bf16),
        ],
    )(emb, x2, x2, dis32, w1t, b1l, w2t, b2l)


def _layer_tc_last(emb, x2, dis32, w1t, b1l, w2t, b2l, e0, e1, e2):
    def body(e_ref, xa_ref, xb_ref, dis_ref, w1_ref, b1_ref, w2_ref, b2_ref,
             e0_ref, e1_ref, e2_ref, all_ref):
        o = _layer_math(e_ref[...], xa_ref[0], xb_ref[0], dis_ref[...],
                        w1_ref[...], b1_ref[...], w2_ref[...], b2_ref[...])
        all_ref[...] = jnp.concatenate(
            [e0_ref[...], e1_ref[...], e2_ref[...], o], axis=1)

    return pl.pallas_call(
        body,
        grid=(GRID,),
        in_specs=_LAYER_SPECS + [
            pl.BlockSpec((TB, DD), lambda i: (i, 0)),
            pl.BlockSpec((TB, DD), lambda i: (i, 0)),
            pl.BlockSpec((TB, DD), lambda i: (i, 0)),
        ],
        out_specs=pl.BlockSpec((TB, 4 * DD), lambda i: (i, 0)),
        out_shape=jax.ShapeDtypeStruct((NP, 4 * DD), f32),
    )(emb, x2, x2, dis32, w1t, b1l, w2t, b2l, e0, e1, e2)


def kernel(user_emb, item_emb, W1, b1, W2, b2, edge_index):
    edge_index = edge_index.astype(i32)
    row2 = jnp.pad(edge_index[0], (0, EP - EE),
                   constant_values=NN).reshape(EG, SUB)
    col2 = jnp.pad(edge_index[1], (0, EP - EE),
                   constant_values=NN).reshape(EG, SUB)

    emb0 = jnp.concatenate([user_emb, item_emb], axis=0)
    emb0 = jnp.pad(emb0, ((0, NP - NN), (0, 0)))

    degp = _deg_call(row2, col2)
    disp = _dis_tc(degp.reshape(NC, NP // 128, 128))
    dis32 = jnp.broadcast_to(disp.reshape(NP, 1), (NP, DD))
    embbf = _scale_tc(emb0, dis32)

    outs = [emb0]
    emb = emb0
    for l in range(LL - 1):
        x2 = _spmm_call(embbf, row2, col2)
        emb, embbf = _layer_tc(emb, x2, dis32,
                               W1[l].T, b1[l][None], W2[l].T, b2[l][None])
        outs.append(emb)

    x2 = _spmm_call(embbf, row2, col2)
    l = LL - 1
    alle = _layer_tc_last(emb, x2, dis32,
                          W1[l].T, b1[l][None], W2[l].T, b2[l][None],
                          outs[0], outs[1], outs[2])
    return (alle[:NU], alle[NU:NN])


# speedup vs baseline: 28.2353x; 1.0116x over previous
import jax
import jax.numpy as jnp
from jax import lax
from jax.experimental import pallas as pl
from jax.experimental.pallas import tpu as pltpu
from jax.experimental.pallas import tpu_sc as plsc

NU = 60000
NI = 40000
NN = NU + NI
EE = 1600000
DD = 32
LL = 3
NC = 2
NS = 16
LANES = 16
NP = 100352
SL = NP // NS
CO = SL // 16
SUB = 128
KS = 4
CE = KS * SUB
EP = NS * CE * 196
EG = EP // SUB
TB = 2048
GRID = NP // TB

f32 = jnp.float32
bf16 = jnp.bfloat16
i32 = jnp.int32


def _sc_mesh():
    return plsc.VectorSubcoreMesh(
        core_axis_name="c", subcore_axis_name="s",
        num_cores=NC, num_subcores=NS)


_SC_PARAMS = pltpu.CompilerParams(use_tc_tiling_on_sc=False)


def _deg_body(row_hbm, col_hbm, deg_hbm, accum, ridx, cidx, ridx2, cidx2,
              ones, stage, sem0, sem1):
    c = lax.axis_index("c")
    s = lax.axis_index("s")

    def zf(i, carry):
        stage[pl.ds(pl.multiple_of(i * LANES, LANES), LANES)] = (
            jnp.zeros((LANES,), f32))
        return carry
    lax.fori_loop(0, SL // LANES, zf, 0)
    pltpu.sync_copy(stage, accum.at[pl.ds(pl.multiple_of(s * SL, 8), SL)])

    def of(i, carry):
        ones[pl.ds(pl.multiple_of(i * LANES, LANES), LANES)] = (
            jnp.ones((LANES,), f32))
        return carry
    lax.fori_loop(0, SUB // LANES, of, 0)
    plsc.subcore_barrier()

    epc = EP // (NC * NS)
    gbase = (c * NS + s) * (epc // SUB)
    bufs = ((ridx, cidx, sem0), (ridx2, cidx2, sem1))

    def fire(j, buf):
        ri, ci, sem = buf
        goff = pl.multiple_of(gbase + j * KS, KS)
        pltpu.sync_copy(row_hbm.at[pl.ds(goff, KS)], ri)
        pltpu.sync_copy(col_hbm.at[pl.ds(goff, KS)], ci)
        for k in range(KS):
            pltpu.async_copy(ones, accum.at[ri.at[k]], sem, add=True)
            pltpu.async_copy(ones, accum.at[ci.at[k]], sem, add=True)

    def drain(buf):
        ri, ci, sem = buf
        dummy = row_hbm.at[pl.ds(0, KS)]
        pltpu.make_async_copy(dummy, ri, sem).wait()
        pltpu.make_async_copy(dummy, ci, sem).wait()

    def pair(g, carry):
        pl.when(g > 0)(lambda: drain(bufs[0]))
        fire(2 * g, bufs[0])
        pl.when(g > 0)(lambda: drain(bufs[1]))
        fire(2 * g + 1, bufs[1])
        return carry
    lax.fori_loop(0, epc // CE // 2, pair, 0)
    drain(bufs[0])
    drain(bufs[1])
    plsc.subcore_barrier()

    off = pl.multiple_of(s * SL, 8)
    pltpu.sync_copy(accum.at[pl.ds(off, SL)], stage)
    pltpu.sync_copy(stage, deg_hbm.at[c, pl.ds(off, SL)])


def _zero_rows_bf(ref, nrows):
    def body(i, carry):
        ref[i, :] = jnp.zeros((DD,), bf16)
        return carry
    lax.fori_loop(0, nrows, body, 0)


def _spmm_body(embbf_hbm, row_hbm, col_hbm, x_hbm,
               accum, cidx0, ridx0, rows0, cidx1, ridx1, rows1,
               gsem0, gsem1, ssem0, ssem1):
    c = lax.axis_index("c")
    s = lax.axis_index("s")

    _zero_rows_bf(rows0, CO)
    for k in range(SL // CO):
        off = pl.multiple_of(s * SL + k * CO, 8)
        pltpu.sync_copy(rows0.at[pl.ds(0, CO)], accum.at[pl.ds(off, CO)])
    plsc.subcore_barrier()

    epc = EP // (NC * NS)
    gbase = (c * NS + s) * (epc // SUB)
    bufs = ((cidx0, ridx0, rows0, gsem0, ssem0),
            (cidx1, ridx1, rows1, gsem1, ssem1))

    def stage(j, buf):
        cidx, ridx, rows, gsem, _ = buf
        goff = pl.multiple_of(gbase + j * KS, KS)
        pltpu.sync_copy(col_hbm.at[pl.ds(goff, KS)], cidx)
        pltpu.sync_copy(row_hbm.at[pl.ds(goff, KS)], ridx)
        for k in range(KS):
            pltpu.async_copy(embbf_hbm.at[cidx.at[k]],
                             rows.at[pl.ds(k * SUB, SUB)], gsem)

    def drain(buf, which):
        _, _, rows, gsem, ssem = buf
        sem = gsem if which == 0 else ssem
        pltpu.make_async_copy(embbf_hbm.at[pl.ds(0, CE)], rows, sem).wait()

    def scatter(buf):
        _, ridx, rows, _, ssem = buf
        for k in range(KS):
            pltpu.async_copy(rows.at[pl.ds(k * SUB, SUB)],
                             accum.at[ridx.at[k]], ssem, add=True)

    def pair(g, carry):
        pl.when(g > 0)(lambda: drain(bufs[0], 1))
        stage(2 * g, bufs[0])
        pl.when(g > 0)(lambda: drain(bufs[1], 1))
        stage(2 * g + 1, bufs[1])
        drain(bufs[0], 0)
        scatter(bufs[0])
        drain(bufs[1], 0)
        scatter(bufs[1])
        return carry
    lax.fori_loop(0, epc // CE // 2, pair, 0)
    drain(bufs[0], 1)
    drain(bufs[1], 1)
    plsc.subcore_barrier()

    for k in range(SL // CO):
        off = pl.multiple_of(s * SL + k * CO, 8)
        pltpu.sync_copy(accum.at[pl.ds(off, CO)], rows0.at[pl.ds(0, CO)])
        pltpu.sync_copy(rows0.at[pl.ds(0, CO)], x_hbm.at[c, pl.ds(off, CO)])


def _deg_call(row2, col2):
    fn = pl.kernel(
        _deg_body,
        out_type=jax.ShapeDtypeStruct((NC, NP), f32),
        mesh=_sc_mesh(),
        compiler_params=_SC_PARAMS,
        scratch_types=[
            pltpu.VMEM_SHARED((NP,), f32),
            pltpu.VMEM((KS, SUB), i32),
            pltpu.VMEM((KS, SUB), i32),
            pltpu.VMEM((KS, SUB), i32),
            pltpu.VMEM((KS, SUB), i32),
            pltpu.VMEM((SUB,), f32),
            pltpu.VMEM((SL,), f32),
            pltpu.SemaphoreType.DMA,
            pltpu.SemaphoreType.DMA,
        ],
    )
    return fn(row2, col2)


def _spmm_call(embbf, row2, col2):
    fn = pl.kernel(
        _spmm_body,
        out_type=jax.ShapeDtypeStruct((NC, NP, DD), bf16),
        mesh=_sc_mesh(),
        compiler_params=_SC_PARAMS,
        scratch_types=[
            pltpu.VMEM_SHARED((NP, DD), bf16),
            pltpu.VMEM((KS, SUB), i32),
            pltpu.VMEM((KS, SUB), i32),
            pltpu.VMEM((CE, DD), bf16),
            pltpu.VMEM((KS, SUB), i32),
            pltpu.VMEM((KS, SUB), i32),
            pltpu.VMEM((CE, DD), bf16),
            pltpu.SemaphoreType.DMA,
            pltpu.SemaphoreType.DMA,
            pltpu.SemaphoreType.DMA,
            pltpu.SemaphoreType.DMA,
        ],
    )
    return fn(embbf, row2, col2)


def _dis_tc(degp_packed):
    def body(dp_ref, dis_ref):
        dp = dp_ref[...]
        dis_ref[...] = lax.rsqrt(jnp.maximum(dp[0] + dp[1], 1.0))

    return pl.pallas_call(
        body,
        grid=(GRID,),
        in_specs=[pl.BlockSpec((NC, TB // 128, 128),
                               lambda i: (0, i, 0))],
        out_specs=pl.BlockSpec((TB // 128, 128), lambda i: (i, 0)),
        out_shape=jax.ShapeDtypeStruct((NP // 128, 128), f32),
    )(degp_packed)


def _scale_tc(emb0, dis32):
    def body(e_ref, d_ref, es_ref):
        es_ref[...] = (e_ref[...] * d_ref[...]).astype(bf16)

    return pl.pallas_call(
        body,
        grid=(GRID,),
        in_specs=[
            pl.BlockSpec((TB, DD), lambda i: (i, 0)),
            pl.BlockSpec((TB, DD), lambda i: (i, 0)),
        ],
        out_specs=pl.BlockSpec((TB, DD), lambda i: (i, 0)),
        out_shape=jax.ShapeDtypeStruct((NP, DD), bf16),
    )(emb0, dis32)


def _layer_math(e, xa, xb, dis_blk, w1, b1, w2, b2):
    x = (xa.astype(f32) + xb.astype(f32)) * dis_blk
    h = (jnp.dot(e + x, w1, preferred_element_type=f32) + b1
         + jnp.dot(x * e, w2, preferred_element_type=f32) + b2)
    a = jnp.where(h >= 0, h, 0.2 * h)
    nrm = jnp.maximum(jnp.sqrt(jnp.sum(a * a, axis=1, keepdims=True)), 1e-12)
    return a / nrm


_LAYER_SPECS = [
    pl.BlockSpec((TB, DD), lambda i: (i, 0)),
    pl.BlockSpec((1, TB, DD), lambda i: (0, i, 0)),
    pl.BlockSpec((1, TB, DD), lambda i: (1, i, 0)),
    pl.BlockSpec((TB, DD), lambda i: (i, 0)),
    pl.BlockSpec((DD, DD), lambda i: (0, 0)),
    pl.BlockSpec((1, DD), lambda i: (0, 0)),
    pl.BlockSpec((DD, DD), lambda i: (0, 0)),
    pl.BlockSpec((1, DD), lambda i: (0, 0)),
]


def _layer_tc(emb, x2, dis32, w1t, b1l, w2t, b2l):
    def body(e_ref, xa_ref, xb_ref, dis_ref, w1_ref, b1_ref, w2_ref, b2_ref,
             eo_ref, eso_ref):
        dis_blk = dis_ref[...]
        o = _layer_math(e_ref[...], xa_ref[0], xb_ref[0], dis_blk,
                        w1_ref[...], b1_ref[...], w2_ref[...], b2_ref[...])
        eo_ref[...] = o
        eso_ref[...] = (o * dis_blk).astype(bf16)

    return pl.pallas_call(
        body,
        grid=(GRID,),
        in_specs=_LAYER_SPECS,
        out_specs=[
            pl.BlockSpec((TB, DD), lambda i: (i, 0)),
            pl.BlockSpec((TB, DD), lambda i: (i, 0)),
        ],
        out_shape=[
            jax.ShapeDtypeStruct((NP, DD), f32),
            jax.ShapeDtypeStruct((NP, DD), bf16),
        ],
    )(emb, x2, x2, dis32, w1t, b1l, w2t, b2l)


def _layer_tc_last(emb, x2, dis32, w1t, b1l, w2t, b2l, e0, e1, e2):
    def body(e_ref, xa_ref, xb_ref, dis_ref, w1_ref, b1_ref, w2_ref, b2_ref,
             e0_ref, e1_ref, e2_ref, all_ref):
        o = _layer_math(e_ref[...], xa_ref[0], xb_ref[0], dis_ref[...],
                        w1_ref[...], b1_ref[...], w2_ref[...], b2_ref[...])
        all_ref[...] = jnp.concatenate(
            [e0_ref[...], e1_ref[...], e2_ref[...], o], axis=1)

    return pl.pallas_call(
        body,
        grid=(GRID,),
        in_specs=_LAYER_SPECS + [
            pl.BlockSpec((TB, DD), lambda i: (i, 0)),
            pl.BlockSpec((TB, DD), lambda i: (i, 0)),
            pl.BlockSpec((TB, DD), lambda i: (i, 0)),
        ],
        out_specs=pl.BlockSpec((TB, 4 * DD), lambda i: (i, 0)),
        out_shape=jax.ShapeDtypeStruct((NP, 4 * DD), f32),
    )(emb, x2, x2, dis32, w1t, b1l, w2t, b2l, e0, e1, e2)


def kernel(user_emb, item_emb, W1, b1, W2, b2, edge_index):
    edge_index = edge_index.astype(i32)
    row2 = jnp.pad(edge_index[0], (0, EP - EE),
                   constant_values=NN).reshape(EG, SUB)
    col2 = jnp.pad(edge_index[1], (0, EP - EE),
                   constant_values=NN).reshape(EG, SUB)

    emb0 = jnp.concatenate([user_emb, item_emb], axis=0)
    emb0 = jnp.pad(emb0, ((0, NP - NN), (0, 0)))

    degp = _deg_call(row2, col2)
    disp = _dis_tc(degp.reshape(NC, NP // 128, 128))
    dis32 = jnp.broadcast_to(disp.reshape(NP, 1), (NP, DD))
    embbf = _scale_tc(emb0, dis32)

    outs = [emb0]
    emb = emb0
    for l in range(LL - 1):
        x2 = _spmm_call(embbf, row2, col2)
        emb, embbf = _layer_tc(emb, x2, dis32,
                               W1[l].T, b1[l][None], W2[l].T, b2[l][None])
        outs.append(emb)

    x2 = _spmm_call(embbf, row2, col2)
    l = LL - 1
    alle = _layer_tc_last(emb, x2, dis32,
                          W1[l].T, b1[l][None], W2[l].T, b2[l][None],
                          outs[0], outs[1], outs[2])
    return (alle[:NU], alle[NU:NN])
